# baseline (speedup 1.0000x reference)
import functools

import numpy as np

import jax
import jax.numpy as jnp
from jax import lax
from jax.experimental import pallas as pl
from jax.experimental.pallas import tpu as pltpu
from jax.experimental.pallas import tpu_sc as plsc

_N = 10000
_E = 320000
_D = 128
_DH = _D // 2
_EPS = 1e-5

_NC = 2
_NS = 16
_NW = _NC * _NS
_CH = 80

_EPTD = _E // _NW
_NCHD = _EPTD // _CH
_EPTM = _E // _NS
_NCHM = _EPTM // _CH

_NPAD = 10240

_PERM = np.empty((_D,), np.int32)
for _c in range(2):
    for _g in range(2):
        _base = _c * 64 + _g * 32
        for _k in range(16):
            _PERM[_base + 2 * _k] = _base + _k
            _PERM[_base + 2 * _k + 1] = _base + 16 + _k
_RPT = _NPAD // _NS
_ZR = 128


@functools.cache
def _sc_kernels():
    mesh = plsc.VectorSubcoreMesh(core_axis_name="c", subcore_axis_name="s",
                                  num_cores=_NC, num_subcores=_NS)
    deg = _make_deg_kernel(mesh)
    msg = _make_msg_kernel(mesh)
    return deg, msg


def _make_deg_kernel(mesh):
    return functools.partial(
        pl.kernel,
        out_type=jax.ShapeDtypeStruct((_NC, _NPAD, 16), jnp.float32),
        mesh=mesh,
        scratch_types=[
            pltpu.VMEM((_NCHD, _CH), jnp.int32),
            pltpu.VMEM((_NCHD, _CH), jnp.float32),
            pltpu.VMEM((_CH, 16), jnp.float32),
            pltpu.VMEM((_ZR, 16), jnp.float32),
            pltpu.VMEM_SHARED((_NPAD, 16), jnp.float32),
        ],
        compiler_params=pltpu.CompilerParams(use_tc_tiling_on_sc=False),
    )(_deg_body)


def _deg_body(ei_hbm, ew_hbm, out_hbm, didx2, ewb2, val2d, zbuf, deg_sh):
    cid = lax.axis_index("c")
    sid = lax.axis_index("s")
    wid = cid * _NS + sid
    zvec = jnp.zeros((16,), jnp.float32)

    def zrow(i, _):
        zbuf[i, :] = zvec
        return 0

    lax.fori_loop(0, _ZR, zrow, 0)
    for j in range(_RPT // _ZR):
        pltpu.sync_copy(zbuf, deg_sh.at[pl.ds(sid * _RPT + j * _ZR, _ZR)])

    pltpu.sync_copy(ei_hbm.at[1, wid], didx2)
    pltpu.sync_copy(ew_hbm.at[wid], ewb2)
    plsc.subcore_barrier()

    def chunk(c, _):
        for g in range(_CH // 16):
            wv = ewb2[c, pl.ds(g * 16, 16)]
            for k in range(16):
                val2d[g * 16 + k, :] = jnp.full((16,), wv[k], jnp.float32)
        pltpu.sync_copy(val2d, deg_sh.at[didx2.at[c]], add=True)
        return 0

    lax.fori_loop(0, _NCHD, chunk, 0)
    plsc.subcore_barrier()
    pltpu.sync_copy(
        deg_sh.at[pl.ds(sid * _RPT, _RPT)],
        out_hbm.at[cid, pl.ds(sid * _RPT, _RPT)],
    )


def _make_msg_kernel(mesh):
    return functools.partial(
        pl.kernel,
        out_type=jax.ShapeDtypeStruct((_NC, _NPAD, _DH), jnp.float32),
        mesh=mesh,
        scratch_types=[
            pltpu.VMEM((_NCHM, _CH), jnp.int32),
            pltpu.VMEM((_NCHM * 2, _CH // 2), jnp.int32),
            pltpu.VMEM((_NCHM, _CH), jnp.float32),
            pltpu.VMEM((_CH, _DH), jnp.bfloat16),
            pltpu.VMEM((_CH, _DH), jnp.bfloat16),
            pltpu.VMEM((_CH, _DH), jnp.float32),
            pltpu.VMEM((_CH, _DH), jnp.float32),
            pltpu.VMEM((_ZR, _DH), jnp.float32),
            pltpu.VMEM_SHARED((_NPAD, _DH), jnp.float32),
            pltpu.SemaphoreType.DMA,
            pltpu.SemaphoreType.DMA,
            pltpu.SemaphoreType.DMA,
            pltpu.SemaphoreType.DMA,
        ],
        compiler_params=pltpu.CompilerParams(use_tc_tiling_on_sc=False,
                                             needs_layout_passes=False),
    )(_msg_body)


def _msg_body(ei_hbm, ei2_hbm, ew_hbm, hp_hbm, out_hbm,
              sidx2, didx2, ewb2, rbf0, rbf1, rf0, rf1, zbuf, acc_sh,
              gsem0, gsem1, ssem0, ssem1):
    cid = lax.axis_index("c")
    sid = lax.axis_index("s")
    zvec = jnp.zeros((16,), jnp.float32)

    def zrow(i, _):
        for f in range(_DH // 16):
            zbuf[i, pl.ds(f * 16, 16)] = zvec
        return 0

    lax.fori_loop(0, _ZR, zrow, 0)
    for j in range(_RPT // _ZR):
        pltpu.sync_copy(zbuf, acc_sh.at[pl.ds(sid * _RPT + j * _ZR, _ZR)])

    pltpu.sync_copy(ei_hbm.at[0, sid], sidx2)
    pltpu.sync_copy(ei2_hbm.at[sid], didx2)
    pltpu.sync_copy(ew_hbm.at[sid], ewb2)
    plsc.subcore_barrier()

    rbf = (rbf0, rbf1)
    rf = (rf0, rf1)
    gsem = (gsem0, gsem1)
    ssem = (ssem0, ssem1)
    table = hp_hbm.at[cid]

    def scale(c, b):
        for g in range(_CH // 16):
            wv = ewb2[c, pl.ds(g * 16, 16)]
            for k in range(16):
                e = g * 16 + k
                w = wv[k]
                for grp in range(_DH // 32):
                    v32 = rbf[b][e, pl.ds(grp * 32, 32)]
                    vi = plsc.bitcast(v32, jnp.int32)
                    lo = plsc.bitcast(lax.shift_left(vi, 16), jnp.float32)
                    hi = plsc.bitcast(
                        jnp.bitwise_and(vi, jnp.int32(-65536)), jnp.float32)
                    rf[b][e, pl.ds(grp * 32, 16)] = lo * w
                    rf[b][e, pl.ds(grp * 32 + 16, 16)] = hi * w

    pltpu.async_copy(table.at[sidx2.at[0]], rbf0, gsem0)
    pltpu.async_copy(table.at[sidx2.at[1]], rbf1, gsem1)
    pltpu.make_async_copy(table.at[sidx2.at[0]], rbf0, gsem0).wait()
    scale(0, 0)
    def scatter(c, b):
        for h in range(2):
            pltpu.async_copy(rf[b].at[pl.ds(h * 40, 40)],
                             acc_sh.at[didx2.at[2 * c + h]], ssem[b],
                             add=True)

    def drain(c, b):
        for h in range(2):
            pltpu.make_async_copy(rf[b].at[pl.ds(h * 40, 40)],
                                  acc_sh.at[didx2.at[2 * c + h]],
                                  ssem[b]).wait()

    scatter(0, 0)
    pltpu.async_copy(table.at[sidx2.at[2]], rbf0, gsem0)
    pltpu.make_async_copy(table.at[sidx2.at[1]], rbf1, gsem1).wait()
    scale(1, 1)
    scatter(1, 1)

    def substep(c, b):
        @pl.when(c + 1 < _NCHM)
        def _():
            pltpu.async_copy(table.at[sidx2.at[c + 1]], rbf[1 - b],
                             gsem[1 - b])
        pltpu.make_async_copy(table.at[sidx2.at[c]], rbf[b], gsem[b]).wait()
        drain(c - 2, b)
        scale(c, b)
        scatter(c, b)

    def pair(j, _):
        substep(2 * j + 2, 0)
        substep(2 * j + 3, 1)
        return 0

    lax.fori_loop(0, (_NCHM - 2) // 2, pair, 0)

    c_last = _NCHM - 1
    drain(c_last - 1, 0)
    drain(c_last, 1)

    plsc.subcore_barrier()
    pltpu.sync_copy(
        acc_sh.at[pl.ds(sid * _RPT, _RPT)],
        out_hbm.at[cid, pl.ds(sid * _RPT, _RPT)],
    )


def _tc_a_body(x_ref, w1_ref, w1p_ref, degp_ref, h1p_ref, self_ref, dis_ref):
    deg = (degp_ref[0, :_N, 0:1] + degp_ref[1, :_N, 0:1]) + 1.0
    dis = lax.rsqrt(deg)
    h = lax.dot_general(x_ref[...], w1_ref[...],
                        (((1,), (1,)), ((), ())),
                        preferred_element_type=jnp.float32)
    hperm = lax.dot_general(x_ref[...], w1p_ref[...],
                            (((1,), (1,)), ((), ())),
                            preferred_element_type=jnp.float32)
    hd = (hperm * dis).astype(jnp.bfloat16)
    h1p_ref[0] = hd[:, :_DH]
    h1p_ref[1] = hd[:, _DH:]
    self_ref[...] = h * (dis * dis)
    dis_ref[...] = dis


def _tc_a(x, W1, W1p, degp):
    return pl.pallas_call(
        _tc_a_body,
        out_shape=(jax.ShapeDtypeStruct((_NC, _N, _DH), jnp.bfloat16),
                   jax.ShapeDtypeStruct((_N, _D), jnp.float32),
                   jax.ShapeDtypeStruct((_N, 1), jnp.float32)),
    )(x, W1, W1p, degp)


def _bn(v, gamma, beta):
    mean = jnp.mean(v, axis=0, keepdims=True)
    var = jnp.mean((v - mean) ** 2, axis=0, keepdims=True)
    return gamma * ((v - mean) * lax.rsqrt(var + _EPS)) + beta


def _tc_b_body(s_ref, self_ref, dis_ref, b1_ref, g1_ref, be1_ref,
               w2_ref, w2p_ref, h2p_ref, self2_ref):
    s_full = jnp.concatenate([s_ref[0, :_N, :], s_ref[1, :_N, :]], axis=1)
    conv = dis_ref[...] * s_full + self_ref[...] + b1_ref[...]
    x1 = jax.nn.gelu(_bn(conv, g1_ref[...], be1_ref[...]))
    dis = dis_ref[...]
    h2 = lax.dot_general(x1, w2_ref[...],
                         (((1,), (1,)), ((), ())),
                         preferred_element_type=jnp.float32)
    h2perm = lax.dot_general(x1, w2p_ref[...],
                             (((1,), (1,)), ((), ())),
                             preferred_element_type=jnp.float32)
    hd = (h2perm * dis).astype(jnp.bfloat16)
    h2p_ref[0] = hd[:, :_DH]
    h2p_ref[1] = hd[:, _DH:]
    self2_ref[...] = h2 * (dis * dis)


def _tc_b(S1, self1, dis, b1, g1, be1, W2, W2p):
    return pl.pallas_call(
        _tc_b_body,
        out_shape=(jax.ShapeDtypeStruct((_NC, _N, _DH), jnp.bfloat16),
                   jax.ShapeDtypeStruct((_N, _D), jnp.float32)),
    )(S1, self1, dis, b1, g1, be1, W2, W2p)


def _tc_c_body(s_ref, self2_ref, dis_ref, b2_ref, g2_ref, be2_ref, x_ref,
               out_ref):
    s_full = jnp.concatenate([s_ref[0, :_N, :], s_ref[1, :_N, :]], axis=1)
    conv = dis_ref[...] * s_full + self2_ref[...] + b2_ref[...]
    out_ref[...] = _bn(conv, g2_ref[...], be2_ref[...]) + x_ref[...]


def _tc_c(S2, self2, dis, b2, g2, be2, x):
    return pl.pallas_call(
        _tc_c_body,
        out_shape=jax.ShapeDtypeStruct((_N, _D), jnp.float32),
    )(S2, self2, dis, b2, g2, be2, x)


def kernel(x, edge_index, edge_weight, W1, b1, g1, be1, W2, b2, g2, be2):
    ei_m = edge_index.reshape(2, _NS, _NCHM, _CH)
    dst_m2 = edge_index[1].reshape(_NS, _NCHM * 2, _CH // 2)
    ew_m = edge_weight.reshape(_NS, _NCHM, _CH)
    ei_d = edge_index.reshape(2, _NW, _NCHD, _CH)
    ew_d = edge_weight.reshape(_NW, _NCHD, _CH)
    b1r = b1.reshape(1, _D)
    g1r = g1.reshape(1, _D)
    be1r = be1.reshape(1, _D)
    b2r = b2.reshape(1, _D)
    g2r = g2.reshape(1, _D)
    be2r = be2.reshape(1, _D)

    W1p = W1[_PERM]
    W2p = W2[_PERM]

    deg_kernel, msg_kernel = _sc_kernels()
    degp = deg_kernel(ei_d, ew_d)
    h1p, self1, dis = _tc_a(x, W1, W1p, degp)
    S1 = msg_kernel(ei_m, dst_m2, ew_m, h1p)
    h2p, self2 = _tc_b(S1, self1, dis, b1r, g1r, be1r, W2, W2p)
    S2 = msg_kernel(ei_m, dst_m2, ew_m, h2p)
    return _tc_c(S2, self2, dis, b2r, g2r, be2r, x)


# speedup vs baseline: 1.0049x; 1.0049x over previous
import functools

import numpy as np

import jax
import jax.numpy as jnp
from jax import lax
from jax.experimental import pallas as pl
from jax.experimental.pallas import tpu as pltpu
from jax.experimental.pallas import tpu_sc as plsc

_N = 10000
_E = 320000
_D = 128
_DH = _D // 2
_EPS = 1e-5

_NC = 2
_NS = 16
_NW = _NC * _NS
_CH = 80

_EPTD = _E // _NW
_NCHD = _EPTD // _CH
_EPTM = _E // _NS
_NCHM = _EPTM // _CH

_NPAD = 10240

_PERM = np.empty((_D,), np.int32)
for _c in range(2):
    for _g in range(2):
        _base = _c * 64 + _g * 32
        for _k in range(16):
            _PERM[_base + 2 * _k] = _base + _k
            _PERM[_base + 2 * _k + 1] = _base + 16 + _k
_RPT = _NPAD // _NS
_ZR = 128


@functools.cache
def _sc_kernels():
    mesh = plsc.VectorSubcoreMesh(core_axis_name="c", subcore_axis_name="s",
                                  num_cores=_NC, num_subcores=_NS)
    deg = _make_deg_kernel(mesh)
    msg = _make_msg_kernel(mesh)
    return deg, msg


def _make_deg_kernel(mesh):
    return functools.partial(
        pl.kernel,
        out_type=jax.ShapeDtypeStruct((_NC, _NPAD, 16), jnp.float32),
        mesh=mesh,
        scratch_types=[
            pltpu.VMEM((_NCHD, _CH), jnp.int32),
            pltpu.VMEM((_NCHD, _CH), jnp.float32),
            pltpu.VMEM((_CH, 16), jnp.float32),
            pltpu.VMEM((_ZR, 16), jnp.float32),
            pltpu.VMEM_SHARED((_NPAD, 16), jnp.float32),
        ],
        compiler_params=pltpu.CompilerParams(use_tc_tiling_on_sc=False),
    )(_deg_body)


def _deg_body(ei_hbm, ew_hbm, out_hbm, didx2, ewb2, val2d, zbuf, deg_sh):
    cid = lax.axis_index("c")
    sid = lax.axis_index("s")
    wid = cid * _NS + sid
    zvec = jnp.zeros((16,), jnp.float32)

    def zrow(i, _):
        zbuf[i, :] = zvec
        return 0

    lax.fori_loop(0, _ZR, zrow, 0)
    for j in range(_RPT // _ZR):
        pltpu.sync_copy(zbuf, deg_sh.at[pl.ds(sid * _RPT + j * _ZR, _ZR)])

    pltpu.sync_copy(ei_hbm.at[1, wid], didx2)
    pltpu.sync_copy(ew_hbm.at[wid], ewb2)
    plsc.subcore_barrier()

    def chunk(c, _):
        for g in range(_CH // 16):
            wv = ewb2[c, pl.ds(g * 16, 16)]
            for k in range(16):
                val2d[g * 16 + k, :] = jnp.full((16,), wv[k], jnp.float32)
        pltpu.sync_copy(val2d, deg_sh.at[didx2.at[c]], add=True)
        return 0

    lax.fori_loop(0, _NCHD, chunk, 0)
    plsc.subcore_barrier()
    pltpu.sync_copy(
        deg_sh.at[pl.ds(sid * _RPT, _RPT)],
        out_hbm.at[cid, pl.ds(sid * _RPT, _RPT)],
    )


def _make_msg_kernel(mesh):
    return functools.partial(
        pl.kernel,
        out_type=jax.ShapeDtypeStruct((_NC, _NPAD, _DH), jnp.float32),
        mesh=mesh,
        scratch_types=[
            pltpu.VMEM((_NCHM, _CH), jnp.int32),
            pltpu.VMEM((_NCHM, _CH), jnp.int32),
            pltpu.VMEM((_NCHM, _CH), jnp.float32),
            pltpu.VMEM((_CH, _DH), jnp.bfloat16),
            pltpu.VMEM((_CH, _DH), jnp.bfloat16),
            pltpu.VMEM((_CH, _DH), jnp.float32),
            pltpu.VMEM((_CH, _DH), jnp.float32),
            pltpu.VMEM((_ZR, _DH), jnp.float32),
            pltpu.VMEM_SHARED((_NPAD, _DH), jnp.float32),
            pltpu.SemaphoreType.DMA,
            pltpu.SemaphoreType.DMA,
            pltpu.SemaphoreType.DMA,
            pltpu.SemaphoreType.DMA,
        ],
        compiler_params=pltpu.CompilerParams(use_tc_tiling_on_sc=False,
                                             needs_layout_passes=False),
    )(_msg_body)


def _msg_body(ei_hbm, ew_hbm, hp_hbm, out_hbm,
              sidx2, didx2, ewb2, rbf0, rbf1, rf0, rf1, zbuf, acc_sh,
              gsem0, gsem1, ssem0, ssem1):
    cid = lax.axis_index("c")
    sid = lax.axis_index("s")
    zvec = jnp.zeros((16,), jnp.float32)

    def zrow(i, _):
        for f in range(_DH // 16):
            zbuf[i, pl.ds(f * 16, 16)] = zvec
        return 0

    lax.fori_loop(0, _ZR, zrow, 0)
    for j in range(_RPT // _ZR):
        pltpu.sync_copy(zbuf, acc_sh.at[pl.ds(sid * _RPT + j * _ZR, _ZR)])

    pltpu.sync_copy(ei_hbm.at[0, sid], sidx2)
    pltpu.sync_copy(ei_hbm.at[1, sid], didx2)
    pltpu.sync_copy(ew_hbm.at[sid], ewb2)
    plsc.subcore_barrier()

    rbf = (rbf0, rbf1)
    rf = (rf0, rf1)
    gsem = (gsem0, gsem1)
    ssem = (ssem0, ssem1)
    table = hp_hbm.at[cid]

    def scale(c, b):
        for g in range(_CH // 16):
            wv = ewb2[c, pl.ds(g * 16, 16)]
            for k in range(16):
                e = g * 16 + k
                w = wv[k]
                for grp in range(_DH // 32):
                    v32 = rbf[b][e, pl.ds(grp * 32, 32)]
                    vi = plsc.bitcast(v32, jnp.int32)
                    lo = plsc.bitcast(lax.shift_left(vi, 16), jnp.float32)
                    hi = plsc.bitcast(
                        jnp.bitwise_and(vi, jnp.int32(-65536)), jnp.float32)
                    rf[b][e, pl.ds(grp * 32, 16)] = lo * w
                    rf[b][e, pl.ds(grp * 32 + 16, 16)] = hi * w

    pltpu.async_copy(table.at[sidx2.at[0]], rbf0, gsem0)
    pltpu.async_copy(table.at[sidx2.at[1]], rbf1, gsem1)
    pltpu.make_async_copy(table.at[sidx2.at[0]], rbf0, gsem0).wait()
    scale(0, 0)
    pltpu.async_copy(rf0, acc_sh.at[didx2.at[0]], ssem0, add=True)
    pltpu.async_copy(table.at[sidx2.at[2]], rbf0, gsem0)
    pltpu.make_async_copy(table.at[sidx2.at[1]], rbf1, gsem1).wait()
    scale(1, 1)
    pltpu.async_copy(rf1, acc_sh.at[didx2.at[1]], ssem1, add=True)

    def substep(c, b):
        @pl.when(c + 1 < _NCHM)
        def _():
            pltpu.async_copy(table.at[sidx2.at[c + 1]], rbf[1 - b],
                             gsem[1 - b])
        pltpu.make_async_copy(table.at[sidx2.at[c]], rbf[b], gsem[b]).wait()
        pltpu.make_async_copy(rf[b], acc_sh.at[didx2.at[c - 2]],
                              ssem[b]).wait()
        scale(c, b)
        pltpu.async_copy(rf[b], acc_sh.at[didx2.at[c]], ssem[b], add=True)

    def pair(j, _):
        substep(2 * j + 2, 0)
        substep(2 * j + 3, 1)
        return 0

    lax.fori_loop(0, (_NCHM - 2) // 2, pair, 0)

    c_last = _NCHM - 1
    pltpu.make_async_copy(rf[0], acc_sh.at[didx2.at[c_last - 1]],
                          ssem[0]).wait()
    pltpu.make_async_copy(rf[1], acc_sh.at[didx2.at[c_last]],
                          ssem[1]).wait()

    plsc.subcore_barrier()
    pltpu.sync_copy(
        acc_sh.at[pl.ds(sid * _RPT, _RPT)],
        out_hbm.at[cid, pl.ds(sid * _RPT, _RPT)],
    )


def _tc_a_body(x_ref, w1_ref, w1p_ref, degp_ref, h1p_ref, self_ref, dis_ref):
    deg = (degp_ref[0, :_N, 0:1] + degp_ref[1, :_N, 0:1]) + 1.0
    dis = lax.rsqrt(deg)
    h = lax.dot_general(x_ref[...], w1_ref[...],
                        (((1,), (1,)), ((), ())),
                        preferred_element_type=jnp.float32)
    hperm = lax.dot_general(x_ref[...], w1p_ref[...],
                            (((1,), (1,)), ((), ())),
                            preferred_element_type=jnp.float32)
    hd = (hperm * dis).astype(jnp.bfloat16)
    h1p_ref[0] = hd[:, :_DH]
    h1p_ref[1] = hd[:, _DH:]
    self_ref[...] = h * (dis * dis)
    dis_ref[...] = dis


def _tc_a(x, W1, W1p, degp):
    return pl.pallas_call(
        _tc_a_body,
        out_shape=(jax.ShapeDtypeStruct((_NC, _N, _DH), jnp.bfloat16),
                   jax.ShapeDtypeStruct((_N, _D), jnp.float32),
                   jax.ShapeDtypeStruct((_N, 1), jnp.float32)),
    )(x, W1, W1p, degp)


def _bn(v, gamma, beta):
    mean = jnp.mean(v, axis=0, keepdims=True)
    var = jnp.mean((v - mean) ** 2, axis=0, keepdims=True)
    return gamma * ((v - mean) * lax.rsqrt(var + _EPS)) + beta


def _tc_b_body(s_ref, self_ref, dis_ref, b1_ref, g1_ref, be1_ref,
               w2_ref, w2p_ref, h2p_ref, self2_ref):
    s_full = jnp.concatenate([s_ref[0, :_N, :], s_ref[1, :_N, :]], axis=1)
    conv = dis_ref[...] * s_full + self_ref[...] + b1_ref[...]
    x1 = jax.nn.gelu(_bn(conv, g1_ref[...], be1_ref[...]))
    dis = dis_ref[...]
    h2 = lax.dot_general(x1, w2_ref[...],
                         (((1,), (1,)), ((), ())),
                         preferred_element_type=jnp.float32)
    h2perm = lax.dot_general(x1, w2p_ref[...],
                             (((1,), (1,)), ((), ())),
                             preferred_element_type=jnp.float32)
    hd = (h2perm * dis).astype(jnp.bfloat16)
    h2p_ref[0] = hd[:, :_DH]
    h2p_ref[1] = hd[:, _DH:]
    self2_ref[...] = h2 * (dis * dis)


def _tc_b(S1, self1, dis, b1, g1, be1, W2, W2p):
    return pl.pallas_call(
        _tc_b_body,
        out_shape=(jax.ShapeDtypeStruct((_NC, _N, _DH), jnp.bfloat16),
                   jax.ShapeDtypeStruct((_N, _D), jnp.float32)),
    )(S1, self1, dis, b1, g1, be1, W2, W2p)


def _tc_c_body(s_ref, self2_ref, dis_ref, b2_ref, g2_ref, be2_ref, x_ref,
               out_ref):
    s_full = jnp.concatenate([s_ref[0, :_N, :], s_ref[1, :_N, :]], axis=1)
    conv = dis_ref[...] * s_full + self2_ref[...] + b2_ref[...]
    out_ref[...] = _bn(conv, g2_ref[...], be2_ref[...]) + x_ref[...]


def _tc_c(S2, self2, dis, b2, g2, be2, x):
    return pl.pallas_call(
        _tc_c_body,
        out_shape=jax.ShapeDtypeStruct((_N, _D), jnp.float32),
    )(S2, self2, dis, b2, g2, be2, x)


def kernel(x, edge_index, edge_weight, W1, b1, g1, be1, W2, b2, g2, be2):
    ei_m = edge_index.reshape(2, _NS, _NCHM, _CH)
    ew_m = edge_weight.reshape(_NS, _NCHM, _CH)
    ei_d = edge_index.reshape(2, _NW, _NCHD, _CH)
    ew_d = edge_weight.reshape(_NW, _NCHD, _CH)
    b1r = b1.reshape(1, _D)
    g1r = g1.reshape(1, _D)
    be1r = be1.reshape(1, _D)
    b2r = b2.reshape(1, _D)
    g2r = g2.reshape(1, _D)
    be2r = be2.reshape(1, _D)

    W1p = W1[_PERM]
    W2p = W2[_PERM]

    deg_kernel, msg_kernel = _sc_kernels()
    degp = deg_kernel(ei_d, ew_d)
    h1p, self1, dis = _tc_a(x, W1, W1p, degp)
    S1 = msg_kernel(ei_m, ew_m, h1p)
    h2p, self2 = _tc_b(S1, self1, dis, b1r, g1r, be1r, W2, W2p)
    S2 = msg_kernel(ei_m, ew_m, h2p)
    return _tc_c(S2, self2, dis, b2r, g2r, be2r, x)
